# baseline (speedup 1.0000x reference)
import jax
import jax.numpy as jnp
from jax import lax
from jax.experimental import pallas as pl
from jax.experimental.pallas import tpu as pltpu
from jax.experimental.pallas import tpu_sc as plsc

N = 50000
E = 800000
NG = 512

NT = 32
NPPAD = 50176
PCH = NPPAD // NT


def _make_pool():
  mesh = plsc.VectorSubcoreMesh(core_axis_name="c", subcore_axis_name="s")
  out_type = [jax.ShapeDtypeStruct((NT, NG), jnp.float32)]
  scratch = [
      pltpu.VMEM((PCH + 16,), jnp.float32),
      pltpu.VMEM((PCH + 16,), jnp.int32),
      pltpu.VMEM((NG + 16,), jnp.float32),
      pltpu.SemaphoreType.DMA,
  ]

  def body(v_hbm, b_hbm, part_hbm, vbuf, bbuf, acc, sem):
    cid = lax.axis_index("c")
    sid = lax.axis_index("s")
    tl = sid * 2 + cid
    i16 = lax.iota(jnp.int32, 16)
    zf = jnp.zeros((16,), jnp.float32)
    for q in range(NG // 16):
      acc[pl.ds(q * 16, 16)] = zf
    acc[pl.ds(NG, 16)] = zf
    pltpu.sync_copy(v_hbm.at[pl.ds(tl * PCH, PCH)], vbuf.at[pl.ds(0, PCH)])
    pltpu.sync_copy(b_hbm.at[pl.ds(tl * PCH, PCH)], bbuf.at[pl.ds(0, PCH)])

    def node_body(n, _):
      g = bbuf[pl.ds(n, 16)][0]
      vn = vbuf[pl.ds(n, 16)][0]
      acc[pl.ds(g, 16)] = acc[pl.ds(g, 16)] + jnp.where(
          i16 < 1, jnp.broadcast_to(vn, (16,)), zf)
      return 0
    lax.fori_loop(0, PCH, node_body, 0)
    pltpu.sync_copy(acc.at[pl.ds(0, NG)], part_hbm.at[tl])

  return pl.kernel(body, out_type=out_type, mesh=mesh, scratch_types=scratch)


BPRE = 2000
BPOST = 1000


def _pre_kernel(x_ref, w_ref, o_ref):
  o_ref[...] = jnp.dot(x_ref[...], w_ref[...],
                       preferred_element_type=jnp.float32)


def _mid_kernel(acc_ref, t1_ref, b1_ref, e64_ref, w2_ref, o_ref):
  accs = acc_ref[...]
  aa = t1_ref[:, 64:72] + t1_ref[:, 72:80]
  wself = jnp.exp(jnp.where(aa > 0, aa, aa * 0.2))
  e64 = e64_ref[...]
  num = accs[:, 0:64] + jnp.dot(wself, e64) * t1_ref[:, 0:64]
  den64 = jnp.dot(accs[:, 64:72] + wself, e64) + 1e-16
  y = num / den64 + b1_ref[...]
  y = jnp.where(y > 0, y, jnp.exp(y) - 1.0)
  o_ref[...] = jnp.dot(y, w2_ref[...], preferred_element_type=jnp.float32)


def _post_kernel(acc_ref, t2_ref, b2_ref, wg_ref, v_ref):
  accs = acc_ref[...]
  aa = t2_ref[:, 128:129] + t2_ref[:, 129:130]
  wself = jnp.exp(jnp.where(aa > 0, aa, aa * 0.2))
  x4 = (accs[:, 0:128] + wself * t2_ref[:, 0:128]) / (
      accs[:, 128:129] + wself + 1e-16)
  x4 = x4 + b2_ref[...]
  v_ref[...] = jnp.dot(x4, wg_ref[...], preferred_element_type=jnp.float32)


def _zsum_kernel(part_ref, bg_ref, z_ref):
  ones = jnp.ones((NT, 1), jnp.float32)
  z_ref[...] = lax.dot_general(part_ref[...], ones, (((0,), (0,)), ((), ())),
                               preferred_element_type=jnp.float32) + bg_ref[...]


def kernel(x, edge_index, batch, W1, att_src1, att_dst1, b1,
           W2, att_src2, att_dst2, b2, Wg, bg):
  f32 = jnp.float32
  a_s1 = (jnp.eye(8, dtype=f32)[:, None, :] *
          att_src1[:, :, None]).reshape(64, 8)
  a_d1 = (jnp.eye(8, dtype=f32)[:, None, :] *
          att_dst1[:, :, None]).reshape(64, 8)
  wcat1 = jnp.concatenate(
      [W1, W1 @ a_s1, W1 @ a_d1, jnp.zeros((75, 48), f32)], axis=1)
  wcat2 = jnp.concatenate(
      [W2, W2 @ att_src2.T, W2 @ att_dst2.T,
       jnp.zeros((64, 126), f32)], axis=1)
  e64 = jnp.kron(jnp.eye(8, dtype=f32), jnp.ones((1, 8), f32))
  src = edge_index[0]
  dst = edge_index[1]

  t1 = pl.pallas_call(
      _pre_kernel,
      grid=(N // BPRE,),
      in_specs=[pl.BlockSpec((BPRE, 75), lambda i: (i, 0)),
                pl.BlockSpec((75, 128), lambda i: (0, 0))],
      out_specs=pl.BlockSpec((BPRE, 128), lambda i: (i, 0)),
      out_shape=jax.ShapeDtypeStruct((N, 128), f32),
  )(x, wcat1)

  a1 = jax.nn.leaky_relu(t1[src, 64:72] + t1[dst, 72:80], 0.2)
  w1e = jnp.exp(a1)
  msg1 = (t1[src, 0:64].reshape(E, 8, 8) * w1e[:, :, None]).reshape(E, 64)
  acc1 = jnp.concatenate([msg1, w1e, jnp.zeros((E, 56), f32)], axis=1)
  acc1 = jax.ops.segment_sum(acc1, dst, num_segments=N)

  t2 = pl.pallas_call(
      _mid_kernel,
      grid=(N // BPRE,),
      in_specs=[pl.BlockSpec((BPRE, 128), lambda i: (i, 0)),
                pl.BlockSpec((BPRE, 128), lambda i: (i, 0)),
                pl.BlockSpec((1, 64), lambda i: (0, 0)),
                pl.BlockSpec((8, 64), lambda i: (0, 0)),
                pl.BlockSpec((64, 256), lambda i: (0, 0))],
      out_specs=pl.BlockSpec((BPRE, 256), lambda i: (i, 0)),
      out_shape=jax.ShapeDtypeStruct((N, 256), f32),
  )(acc1, t1, b1.reshape(1, 64), e64, wcat2)

  a2 = jax.nn.leaky_relu(t2[src, 128] + t2[dst, 129], 0.2)
  w2e = jnp.exp(a2)
  msg2 = t2[src, 0:128] * w2e[:, None]
  acc2 = jnp.concatenate([msg2, w2e[:, None], jnp.zeros((E, 127), f32)],
                         axis=1)
  acc2 = jax.ops.segment_sum(acc2, dst, num_segments=N)

  v = pl.pallas_call(
      _post_kernel,
      grid=(N // BPOST,),
      in_specs=[pl.BlockSpec((BPOST, 256), lambda i: (i, 0)),
                pl.BlockSpec((BPOST, 256), lambda i: (i, 0)),
                pl.BlockSpec((1, 128), lambda i: (0, 0)),
                pl.BlockSpec((128, 1), lambda i: (0, 0))],
      out_specs=pl.BlockSpec((BPOST, 1), lambda i: (i, 0)),
      out_shape=jax.ShapeDtypeStruct((N, 1), f32),
  )(acc2, t2, b2.reshape(1, 128), Wg)

  vpad = jnp.concatenate([v[:, 0], jnp.zeros((NPPAD - N,), f32)])
  bpad = jnp.concatenate([batch, jnp.zeros((NPPAD - N,), jnp.int32)])
  (part,) = _make_pool()(vpad, bpad)

  z = pl.pallas_call(
      _zsum_kernel,
      in_specs=[pl.BlockSpec((NT, NG), lambda: (0, 0)),
                pl.BlockSpec((1, 1), lambda: (0, 0))],
      out_specs=pl.BlockSpec((NG, 1), lambda: (0, 0)),
      out_shape=jax.ShapeDtypeStruct((NG, 1), f32),
  )(part, bg.reshape(1, 1))
  return z
